# baseline (speedup 1.0000x reference)
import functools

import numpy as np
import jax
import jax.numpy as jnp
from jax import lax
from jax.experimental import pallas as pl
from jax.experimental.pallas import tpu as pltpu
from jax.experimental.pallas import tpu_sc as plsc

NUM_CLIPS = 10
CLIP_LEN = 100
THRESHOLD = 0.01
W = 960.0
H = 576.0
NUM_PERSON = 2
V = 17
C = 3
D = V * C
LANES = 128
CLIP_W = CLIP_LEN * D
CLIP_PAD = 5120
ROWS_PER_CLIP = 41
NW = 32


def _rotl(x, n):
    return ((x << np.uint32(n)) | (x >> np.uint32(32 - n))).astype(np.uint32)


def _tf2x32(k0, k1, x0, x1):
    ks0 = np.uint32(k0)
    ks1 = np.uint32(k1)
    ks2 = np.uint32(ks0 ^ ks1 ^ np.uint32(0x1BD11BDA))
    x0 = (x0 + ks0).astype(np.uint32)
    x1 = (x1 + ks1).astype(np.uint32)
    rot = [(13, 15, 26, 6), (17, 29, 16, 24)]
    ks = [ks0, ks1, ks2]
    for i in range(5):
        for r in rot[i % 2]:
            x0 = (x0 + x1).astype(np.uint32)
            x1 = _rotl(x1, r)
            x1 = (x1 ^ x0).astype(np.uint32)
        x0 = (x0 + ks[(i + 1) % 3]).astype(np.uint32)
        x1 = (x1 + ks[(i + 2) % 3] + np.uint32(i + 1)).astype(np.uint32)
    return x0, x1


def _tf_split(k0, k1, n):
    b1, b2 = _tf2x32(k0, k1, np.zeros(n, np.uint32),
                     np.arange(n, dtype=np.uint32))
    return np.stack([b1, b2], axis=1)


def _tf_bits32(k0, k1, n):
    b1, b2 = _tf2x32(k0, k1, np.zeros(n, np.uint32),
                     np.arange(n, dtype=np.uint32))
    return (b1 ^ b2).astype(np.uint32)


def _np_starts(B, nclips, T, seed=42):
    k0, k1 = np.uint32(seed >> 32), np.uint32(seed & 0xFFFFFFFF)
    sk = _tf_split(k0, k1, B)
    out = np.zeros((B, nclips), np.int64)
    span = np.uint32(T)
    mult = (np.uint64(65536) % span) ** np.uint64(2) % span
    for b in range(B):
        kk = _tf_split(sk[b, 0], sk[b, 1], 2)
        hi = _tf_bits32(kk[0, 0], kk[0, 1], nclips).astype(np.uint64)
        lo = _tf_bits32(kk[1, 0], kk[1, 1], nclips).astype(np.uint64)
        out[b] = (hi % span * mult + lo % span) % span
    return out


_plan_cache = {}


def _clip_plan(B, T):
    key = (B, T)
    if key in _plan_cache:
        return _plan_cache[key]
    starts = _np_starts(B, NUM_CLIPS, T)
    rows_per_video = T * D // LANES
    e0 = starts * D
    r0 = e0 // LANES
    off = (e0 % LANES).astype(np.int32)
    i = np.arange(ROWS_PER_CLIP)
    rows_local = (r0[..., None] + i) % rows_per_video
    rows = rows_local + (np.arange(B) * rows_per_video)[:, None, None]
    nk = B * NUM_CLIPS
    plan = np.zeros((nk, LANES), np.int32)
    plan[:, :ROWS_PER_CLIP] = rows.reshape(nk, ROWS_PER_CLIP)
    plan[:, 64:80] = np.broadcast_to(off.reshape(nk, 1), (nk, 16))
    res = jnp.asarray(plan)
    _plan_cache[key] = res
    return res


def _norm_body(kp_ref, out_ref):
    nr = kp_ref.shape[1]
    v = kp_ref[0]
    l = lax.broadcasted_iota(jnp.int32, (nr, LANES), 1)
    r = lax.broadcasted_iota(jnp.int32, (nr, LANES), 0)
    ch = (2 * r + l) % 3
    down1 = jnp.roll(v, -1, axis=0)
    a1 = jnp.roll(v, -1, axis=1)
    a2 = jnp.roll(v, -2, axis=1)
    b1 = jnp.roll(down1, -1, axis=1)
    b2 = jnp.roll(down1, -2, axis=1)
    sh1 = jnp.where(l == LANES - 1, b1, a1)
    sh2 = jnp.where(l >= LANES - 2, b2, a2)
    conf = jnp.where(ch == 0, sh2, jnp.where(ch == 1, sh1, v))
    scale = jnp.where(ch == 0, 2.0 / W,
                      jnp.where(ch == 1, 2.0 / H, 1.0)).astype(jnp.float32)
    offs = jnp.where(ch == 2, 0.0, -1.0).astype(jnp.float32)
    out_ref[0] = jnp.where(ch == 2, v,
                           jnp.where(conf <= THRESHOLD, 0.0,
                                     v * scale + offs))


def _normalize(kp):
    B, NR, _ = kp.shape
    return pl.pallas_call(
        _norm_body,
        grid=(B,),
        in_specs=[pl.BlockSpec((1, NR, LANES), lambda b: (b, 0, 0))],
        out_specs=pl.BlockSpec((1, NR, LANES), lambda b: (b, 0, 0)),
        out_shape=jax.ShapeDtypeStruct((B, NR, LANES), jnp.float32),
    )(kp)


def _sc_gather(table, plan, nk):
    cpw = nk // NW

    rows_out = CLIP_PAD // LANES

    def body(table_hbm, plan_hbm, out_hbm, plan_v,
             buf_a, buf_b, sh_a, sh_b, gs_a, gs_b, ss_a, ss_b):
        nc = 2
        wid = lax.axis_index("s") * nc + lax.axis_index("c")
        base = wid * cpw
        pltpu.sync_copy(plan_hbm.at[pl.ds(base, cpw)], plan_v)
        iota = lax.iota(jnp.int32, 16)

        def gather_start(k, buf, gsem):
            idx = plan_v.at[k, pl.ds(0, ROWS_PER_CLIP)]
            return pltpu.async_copy(table_hbm.at[idx], buf, gsem)

        def half(p, k, hnd, buf, sh, ssem):
            hnd.wait()
            offv = plan_v[k, pl.ds(64, 16)]

            @pl.when(p > 0)
            def _():
                pltpu.make_async_copy(
                    sh, out_hbm.at[pl.ds(base * rows_out, rows_out)],
                    ssem).wait()

            def jbody(j, _):
                e = offv + j * 16 + iota
                rr = lax.shift_right_logical(e, 7)
                cc = lax.bitwise_and(e, 127)
                val = plsc.load_gather(buf, [rr, cc])
                sh[j >> 3, pl.ds(pl.multiple_of((j & 7) * 16, 16), 16)] = val
                return 0

            lax.fori_loop(0, CLIP_PAD // 16, jbody, 0, unroll=8)
            pltpu.async_copy(
                sh, out_hbm.at[pl.ds((base + k) * rows_out, rows_out)], ssem)

        def pair(p, _):
            k0 = 2 * p
            k1 = k0 + 1
            h_a = gather_start(k0, buf_a, gs_a)
            h_b = gather_start(k1, buf_b, gs_b)
            half(p, k0, h_a, buf_a, sh_a, ss_a)
            half(p, k1, h_b, buf_b, sh_b, ss_b)
            return 0

        lax.fori_loop(0, cpw // 2, pair, 0)
        for sh, ssem in ((sh_a, ss_a), (sh_b, ss_b)):
            pltpu.make_async_copy(
                sh, out_hbm.at[pl.ds(base * rows_out, rows_out)], ssem).wait()

    mesh = plsc.VectorSubcoreMesh(core_axis_name="c", subcore_axis_name="s")
    f = functools.partial(
        pl.kernel,
        out_type=jax.ShapeDtypeStruct((nk * (CLIP_PAD // LANES), LANES),
                                      jnp.float32),
        mesh=mesh,
        compiler_params=pltpu.CompilerParams(needs_layout_passes=False),
        scratch_types=[
            pltpu.VMEM((cpw, LANES), jnp.int32),
            pltpu.VMEM((ROWS_PER_CLIP, LANES), jnp.float32),
            pltpu.VMEM((ROWS_PER_CLIP, LANES), jnp.float32),
            pltpu.VMEM((CLIP_PAD // LANES, LANES), jnp.float32),
            pltpu.VMEM((CLIP_PAD // LANES, LANES), jnp.float32),
            pltpu.SemaphoreType.DMA,
            pltpu.SemaphoreType.DMA,
            pltpu.SemaphoreType.DMA,
            pltpu.SemaphoreType.DMA,
        ],
    )(body)
    return f(table, plan)


def _strip_body(in_ref, out_ref):
    out_ref[...] = in_ref[:, 0:CLIP_W]


def _strip(rows2, nk):
    g = 64
    return pl.pallas_call(
        _strip_body,
        grid=(nk // g,),
        in_specs=[pl.BlockSpec((g, CLIP_PAD), lambda i: (i, 0))],
        out_specs=pl.BlockSpec((g, CLIP_W), lambda i: (i, 0)),
        out_shape=jax.ShapeDtypeStruct((nk, CLIP_W), jnp.float32),
    )(rows2)


def kernel(keypoints):
    B, T = keypoints.shape[0], keypoints.shape[1]
    nr = T * D // LANES
    kp = keypoints.reshape(B, nr, LANES)
    norm = _normalize(kp)
    table = norm.reshape(B * nr, LANES)
    plan = _clip_plan(B, T)
    nk = B * NUM_CLIPS
    rows = _sc_gather(table, plan, nk)
    clips = _strip(rows.reshape(nk, CLIP_PAD), nk)
    out = clips.reshape(B, NUM_CLIPS, 1, CLIP_LEN, V, C)
    zeros = jnp.zeros_like(out)
    return jnp.concatenate([out, zeros], axis=2)


# speedup vs baseline: 2.7377x; 2.7377x over previous
import functools

import numpy as np
import jax
import jax.numpy as jnp
from jax import lax
from jax.experimental import pallas as pl
from jax.experimental.pallas import tpu as pltpu
from jax.experimental.pallas import tpu_sc as plsc

NUM_CLIPS = 10
CLIP_LEN = 100
THRESHOLD = 0.01
W = 960.0
H = 576.0
NUM_PERSON = 2
V = 17
C = 3
D = V * C
LANES = 128
CLIP_W = CLIP_LEN * D
CLIP_PAD = 5120
ROWS_PER_CLIP = 41
NW = 32
B_SC = 0


def _rotl(x, n):
    return ((x << np.uint32(n)) | (x >> np.uint32(32 - n))).astype(np.uint32)


def _tf2x32(k0, k1, x0, x1):
    ks0 = np.uint32(k0)
    ks1 = np.uint32(k1)
    ks2 = np.uint32(ks0 ^ ks1 ^ np.uint32(0x1BD11BDA))
    x0 = (x0 + ks0).astype(np.uint32)
    x1 = (x1 + ks1).astype(np.uint32)
    rot = [(13, 15, 26, 6), (17, 29, 16, 24)]
    ks = [ks0, ks1, ks2]
    for i in range(5):
        for r in rot[i % 2]:
            x0 = (x0 + x1).astype(np.uint32)
            x1 = _rotl(x1, r)
            x1 = (x1 ^ x0).astype(np.uint32)
        x0 = (x0 + ks[(i + 1) % 3]).astype(np.uint32)
        x1 = (x1 + ks[(i + 2) % 3] + np.uint32(i + 1)).astype(np.uint32)
    return x0, x1


def _tf_split(k0, k1, n):
    b1, b2 = _tf2x32(k0, k1, np.zeros(n, np.uint32),
                     np.arange(n, dtype=np.uint32))
    return np.stack([b1, b2], axis=1)


def _tf_bits32(k0, k1, n):
    b1, b2 = _tf2x32(k0, k1, np.zeros(n, np.uint32),
                     np.arange(n, dtype=np.uint32))
    return (b1 ^ b2).astype(np.uint32)


def _np_starts(B, nclips, T, seed=42):
    k0, k1 = np.uint32(seed >> 32), np.uint32(seed & 0xFFFFFFFF)
    sk = _tf_split(k0, k1, B)
    out = np.zeros((B, nclips), np.int64)
    span = np.uint32(T)
    mult = (np.uint64(65536) % span) ** np.uint64(2) % span
    for b in range(B):
        kk = _tf_split(sk[b, 0], sk[b, 1], 2)
        hi = _tf_bits32(kk[0, 0], kk[0, 1], nclips).astype(np.uint64)
        lo = _tf_bits32(kk[1, 0], kk[1, 1], nclips).astype(np.uint64)
        out[b] = (hi % span * mult + lo % span) % span
    return out


_starts_cache = {}


def _starts(B, T):
    key = (B, T)
    if key not in _starts_cache:
        _starts_cache[key] = _np_starts(B, NUM_CLIPS, T)
    return _starts_cache[key]


def _sc_plan(starts_sc, T, b0):
    rows_per_video = T * D // LANES
    e0 = starts_sc * D
    r0 = e0 // LANES
    off = (e0 % LANES).astype(np.int32)
    i = np.arange(ROWS_PER_CLIP)
    rows_local = (r0[..., None] + i) % rows_per_video
    nb = starts_sc.shape[0]
    rows = rows_local + ((b0 + np.arange(nb)) * rows_per_video)[:, None, None]
    nk = nb * NUM_CLIPS
    plan = np.zeros((nk, LANES), np.int32)
    plan[:, :ROWS_PER_CLIP] = rows.reshape(nk, ROWS_PER_CLIP)
    plan[:, 64:80] = np.broadcast_to(off.reshape(nk, 1), (nk, 16))
    return jnp.asarray(plan)


def _tc_body(starts_ref, kp_ref, out_ref, scratch):
    T = kp_ref.shape[1]
    scratch[0:T, :] = kp_ref[0]
    scratch[T:T + CLIP_LEN, :] = kp_ref[0, 0:CLIP_LEN, :]
    b = pl.program_id(0)
    lane = lax.broadcasted_iota(jnp.int32, (CLIP_LEN, D), 1)
    ch = lane % 3
    scale = jnp.where(ch == 0, 2.0 / W,
                      jnp.where(ch == 1, 2.0 / H, 1.0)).astype(jnp.float32)
    offset = jnp.where(ch == 2, 0.0, -1.0).astype(jnp.float32)
    for c in range(NUM_CLIPS):
        start = starts_ref[b * NUM_CLIPS + c]
        v = scratch[pl.ds(start, CLIP_LEN), :]
        conf = jnp.where(ch == 0, jnp.roll(v, -2, axis=1),
                         jnp.where(ch == 1, jnp.roll(v, -1, axis=1), v))
        nv = v * scale + offset
        out_ref[0, c] = jnp.where(ch == 2, v,
                                  jnp.where(conf <= THRESHOLD, 0.0, nv))


def _tc_gather(kp, starts):
    B, T, _ = kp.shape
    return pl.pallas_call(
        _tc_body,
        grid_spec=pltpu.PrefetchScalarGridSpec(
            num_scalar_prefetch=1,
            grid=(B,),
            in_specs=[pl.BlockSpec((1, T, D), lambda b, s: (b, 0, 0))],
            out_specs=pl.BlockSpec((1, NUM_CLIPS, CLIP_LEN, D),
                                   lambda b, s: (b, 0, 0, 0)),
            scratch_shapes=[pltpu.VMEM((T + CLIP_LEN, D), jnp.float32)],
        ),
        out_shape=jax.ShapeDtypeStruct((B, NUM_CLIPS, CLIP_LEN, D),
                                       jnp.float32),
    )(starts, kp)


def _norm_body(kp_ref, out_ref):
    nr = kp_ref.shape[1]
    v = kp_ref[0]
    l = lax.broadcasted_iota(jnp.int32, (nr, LANES), 1)
    r = lax.broadcasted_iota(jnp.int32, (nr, LANES), 0)
    ch = (2 * r + l) % 3
    down1 = jnp.roll(v, -1, axis=0)
    a1 = jnp.roll(v, -1, axis=1)
    a2 = jnp.roll(v, -2, axis=1)
    b1 = jnp.roll(down1, -1, axis=1)
    b2 = jnp.roll(down1, -2, axis=1)
    sh1 = jnp.where(l == LANES - 1, b1, a1)
    sh2 = jnp.where(l >= LANES - 2, b2, a2)
    conf = jnp.where(ch == 0, sh2, jnp.where(ch == 1, sh1, v))
    scale = jnp.where(ch == 0, 2.0 / W,
                      jnp.where(ch == 1, 2.0 / H, 1.0)).astype(jnp.float32)
    offs = jnp.where(ch == 2, 0.0, -1.0).astype(jnp.float32)
    out_ref[0] = jnp.where(ch == 2, v,
                           jnp.where(conf <= THRESHOLD, 0.0,
                                     v * scale + offs))


def _normalize(kp):
    B, NR, _ = kp.shape
    return pl.pallas_call(
        _norm_body,
        grid=(B,),
        in_specs=[pl.BlockSpec((1, NR, LANES), lambda b: (b, 0, 0))],
        out_specs=pl.BlockSpec((1, NR, LANES), lambda b: (b, 0, 0)),
        out_shape=jax.ShapeDtypeStruct((B, NR, LANES), jnp.float32),
    )(kp)


def _sc_gather(table, plan, nk):
    cpw = nk // NW
    rows_out = CLIP_PAD // LANES

    def body(table_hbm, plan_hbm, out_hbm, plan_v,
             buf_a, buf_b, sh_a, sh_b, gs_a, gs_b, ss_a, ss_b):
        nc = 2
        wid = lax.axis_index("s") * nc + lax.axis_index("c")
        base = wid * cpw
        pltpu.sync_copy(plan_hbm.at[pl.ds(base, cpw)], plan_v)
        iota = lax.iota(jnp.int32, 16)

        def gather_start(k, buf, gsem):
            idx = plan_v.at[k, pl.ds(0, ROWS_PER_CLIP)]
            return pltpu.async_copy(table_hbm.at[idx], buf, gsem)

        def half(p, k, hnd, buf, sh, ssem):
            hnd.wait()
            offv = plan_v[k, pl.ds(64, 16)]

            @pl.when(p > 0)
            def _():
                pltpu.make_async_copy(
                    sh, out_hbm.at[pl.ds(base * rows_out, rows_out)],
                    ssem).wait()

            def jbody(j, _):
                e = offv + j * 16 + iota
                rr = lax.shift_right_logical(e, 7)
                cc = lax.bitwise_and(e, 127)
                val = plsc.load_gather(buf, [rr, cc])
                sh[j >> 3, pl.ds(pl.multiple_of((j & 7) * 16, 16), 16)] = val
                return 0

            lax.fori_loop(0, CLIP_PAD // 16, jbody, 0, unroll=8)
            pltpu.async_copy(
                sh, out_hbm.at[pl.ds((base + k) * rows_out, rows_out)], ssem)

        def pair(p, _):
            k0 = 2 * p
            k1 = k0 + 1
            h_a = gather_start(k0, buf_a, gs_a)
            h_b = gather_start(k1, buf_b, gs_b)
            half(p, k0, h_a, buf_a, sh_a, ss_a)
            half(p, k1, h_b, buf_b, sh_b, ss_b)
            return 0

        lax.fori_loop(0, cpw // 2, pair, 0)
        for sh, ssem in ((sh_a, ss_a), (sh_b, ss_b)):
            pltpu.make_async_copy(
                sh, out_hbm.at[pl.ds(base * rows_out, rows_out)], ssem).wait()

    mesh = plsc.VectorSubcoreMesh(core_axis_name="c", subcore_axis_name="s")
    f = functools.partial(
        pl.kernel,
        out_type=jax.ShapeDtypeStruct((nk * rows_out, LANES), jnp.float32),
        mesh=mesh,
        compiler_params=pltpu.CompilerParams(needs_layout_passes=False),
        scratch_types=[
            pltpu.VMEM((cpw, LANES), jnp.int32),
            pltpu.VMEM((ROWS_PER_CLIP, LANES), jnp.float32),
            pltpu.VMEM((ROWS_PER_CLIP, LANES), jnp.float32),
            pltpu.VMEM((rows_out, LANES), jnp.float32),
            pltpu.VMEM((rows_out, LANES), jnp.float32),
            pltpu.SemaphoreType.DMA,
            pltpu.SemaphoreType.DMA,
            pltpu.SemaphoreType.DMA,
            pltpu.SemaphoreType.DMA,
        ],
    )(body)
    return f(table, plan)


def kernel(keypoints):
    B, T = keypoints.shape[0], keypoints.shape[1]
    starts = _starts(B, T)
    outs = []
    if B_SC:
        kp_sc = keypoints[:B_SC].reshape(B_SC, T * D // LANES, LANES)
        norm = _normalize(kp_sc)
        table = norm.reshape(B_SC * (T * D // LANES), LANES)
        plan = _sc_plan(starts[:B_SC], T, 0)
        nk = B_SC * NUM_CLIPS
        rows = _sc_gather(table, plan, nk)
        clips = rows.reshape(nk, CLIP_PAD)[:, :CLIP_W]
        outs.append(clips.reshape(B_SC, NUM_CLIPS, CLIP_LEN, D))
    if B_SC < B:
        kp_tc = keypoints[B_SC:].reshape(B - B_SC, T, D)
        st = jnp.asarray(starts[B_SC:].reshape(-1).astype(np.int32))
        outs.append(_tc_gather(kp_tc, st))
    out_c = outs[0] if len(outs) == 1 else jnp.concatenate(outs, axis=0)
    out = out_c.reshape(B, NUM_CLIPS, 1, CLIP_LEN, V, C)
    zeros = jnp.zeros_like(out)
    return jnp.concatenate([out, zeros], axis=2)
